# Initial kernel scaffold; baseline (speedup 1.0000x reference)
import jax
import jax.numpy as jnp
from jax.experimental import pallas as pl


def kernel(inputs, kernel):
    raise NotImplementedError("write your pallas kernel here")


# speedup vs baseline: 35.7446x; 35.7446x over previous
import functools
import itertools

import jax
import jax.numpy as jnp
import numpy as np
from jax.experimental import pallas as pl
from jax.experimental.pallas import tpu as pltpu

NUM_BITS = 32
ORDER = 3
BATCH = 16384


def _combos(k):
    return np.array(list(itertools.combinations(range(NUM_BITS), k)), dtype=np.int32)


_C1 = _combos(1)
_C2 = _combos(2)
_C3 = _combos(3)
_N1, _N2, _N3 = _C1.shape[0], _C2.shape[0], _C3.shape[0]

_NCOL_USED = _N2 + _N1
_NCOL = 640

_PAIR_COL = {(int(j), int(k)): p for p, (j, k) in enumerate(_C2)}

_EJ = np.zeros((NUM_BITS, _NCOL), np.float32)
_EK = np.zeros((NUM_BITS, _NCOL), np.float32)
_W3SEL = np.zeros((NUM_BITS, _NCOL), np.float32)
_W3ROW = np.zeros(_NCOL, np.int32)

for p, (j, k) in enumerate(_C2):
    _EJ[j, p] = 1.0
    _EK[k, p] = 1.0
for q, (i,) in enumerate(_C1):
    col = _N2 + q
    _EJ[i, col] = 1.0
    _EK[i, col] = 1.0

_W3_ROWS = _C3[:, 0]
_W3_COLS = np.array([_PAIR_COL[(int(j), int(k))] for (_, j, k) in _C3], np.int32)
_W1_ROWS = _C1[:, 0]
_W1_COLS = np.arange(_N2, _N2 + _N1, dtype=np.int32)


def _build_constants(w):
    w1 = w[:_N1]
    w2 = w[_N1:_N1 + _N2]
    w3 = w[_N1 + _N2:]
    w3mat = jnp.zeros((NUM_BITS, _NCOL), jnp.float32)
    w3mat = w3mat.at[_W3_ROWS, _W3_COLS].set(w3)
    w3mat = w3mat.at[_W1_ROWS, _W1_COLS].set(w1)
    w2pad = jnp.zeros((1, _NCOL), jnp.float32).at[0, :_N2].set(w2)
    cm = jnp.concatenate(
        [jnp.asarray(_EJ), jnp.asarray(_EK), w3mat], axis=1
    )
    return cm, w2pad


_BBLK = 2048


def _energy_body(x_ref, cm_ref, w2_ref, out_ref):
    s = (1 - 2 * x_ref[...]).astype(jnp.float32)
    prod = jax.lax.dot_general(
        s, cm_ref[...], (((1,), (0,)), ((), ())),
        preferred_element_type=jnp.float32,
    )
    sj = prod[:, :_NCOL]
    sk = prod[:, _NCOL:2 * _NCOL]
    a2 = prod[:, 2 * _NCOL:] + w2_ref[...]
    out_ref[...] = jnp.sum(sj * sk * a2, axis=1, keepdims=True)


@jax.jit
def kernel(inputs, kernel):
    cm, w2pad = _build_constants(kernel)
    grid = BATCH // _BBLK
    out = pl.pallas_call(
        _energy_body,
        grid=(grid,),
        in_specs=[
            pl.BlockSpec((_BBLK, NUM_BITS), lambda i: (i, 0)),
            pl.BlockSpec((NUM_BITS, 3 * _NCOL), lambda i: (0, 0)),
            pl.BlockSpec((1, _NCOL), lambda i: (0, 0)),
        ],
        out_specs=pl.BlockSpec((_BBLK, 1), lambda i: (i, 0)),
        out_shape=jax.ShapeDtypeStruct((BATCH, 1), jnp.float32),
        compiler_params=pltpu.CompilerParams(
            dimension_semantics=("arbitrary",),
        ),
    )(inputs, cm, w2pad)
    return out[:, 0]
